# baseline (speedup 1.0000x reference)
import functools

import jax
import jax.numpy as jnp
from jax import lax
from jax.experimental import pallas as pl
from jax.experimental.pallas import tpu as pltpu
from jax.experimental.pallas import tpu_sc as plsc

_E = 8
_K = 2
_D = 768
_DFF = 2048
_T = 2048
_TK = _T * _K

_BT = 128
_NT = _TK // _BT + _E
_PMAX = _NT * _BT

_NC, _NS = 2, 16
_NW = _NC * _NS

_W_ROWS = _T // _NW
_SLOTS_W = _K * _W_ROWS

_RR = _TK // 128


def _route_body(ids_ref, pos_ref, te_ref, va_ref):
    ids = ids_ref[...]
    jj = lax.broadcasted_iota(jnp.int32, (128, 128), 0)
    ll = lax.broadcasted_iota(jnp.int32, (128, 128), 1)
    tri_lane = (jj <= ll).astype(jnp.float32)
    ii = lax.broadcasted_iota(jnp.int32, (_RR, _RR), 0)
    kk = lax.broadcasted_iota(jnp.int32, (_RR, _RR), 1)
    tri_row = (kk < ii).astype(jnp.float32)

    ohs, csums, counts = [], [], []
    for e in range(_E):
        oh = (ids == e).astype(jnp.float32)
        lane_cs = jnp.dot(oh, tri_lane, preferred_element_type=jnp.float32)
        row_tot = lane_cs[:, 127:128]
        row_pre = jnp.dot(tri_row, row_tot, preferred_element_type=jnp.float32)
        ohs.append(oh)
        csums.append(lane_cs + row_pre)
        counts.append(jnp.sum(row_tot))

    poffs, cps = [], []
    cum = jnp.float32(0.0)
    for e in range(_E):
        poffs.append(cum)
        cum = cum + jnp.ceil(counts[e] * (1.0 / _BT)) * _BT
        cps.append(cum)

    posf = jnp.zeros((_RR, 128), jnp.float32)
    for e in range(_E):
        posf = posf + ohs[e] * (poffs[e] + csums[e] - 1.0)
    pos_ref[...] = posf.astype(jnp.int32)

    starts = (lax.broadcasted_iota(jnp.int32, (1, 128), 1)
              .astype(jnp.float32) * float(_BT))
    te = jnp.zeros((1, 128), jnp.int32)
    for e in range(_E):
        te = te + (starts >= cps[e]).astype(jnp.int32)
    te_ref[...] = jnp.minimum(te, _E - 1)
    va_ref[...] = (starts < cum).astype(jnp.int32)


def _tc_route(ids_grid):
    return pl.pallas_call(
        _route_body,
        out_shape=(
            jax.ShapeDtypeStruct((_RR, 128), jnp.int32),
            jax.ShapeDtypeStruct((1, 128), jnp.int32),
            jax.ShapeDtypeStruct((1, 128), jnp.int32),
        ),
    )(ids_grid)


def _dispatch_body(x_hbm, pos_hbm, xs_hbm, xv, i0_v, i1_v, s0, s1, s2):
    wid = lax.axis_index("s") * _NC + lax.axis_index("c")
    b = wid * _W_ROWS
    l0 = pltpu.async_copy(pos_hbm.at[pl.ds(b, _W_ROWS)], i0_v, s0)
    l1 = pltpu.async_copy(pos_hbm.at[pl.ds(_T + b, _W_ROWS)], i1_v, s1)
    lx = pltpu.async_copy(x_hbm.at[pl.ds(b, _W_ROWS)], xv, s2)
    l0.wait()
    l1.wait()
    lx.wait()
    c0 = pltpu.async_copy(xv, xs_hbm.at[i0_v], s0)
    c1 = pltpu.async_copy(xv, xs_hbm.at[i1_v], s1)
    c0.wait()
    c1.wait()


def _sc_dispatch(x, pos):
    mesh = plsc.VectorSubcoreMesh(core_axis_name="c", subcore_axis_name="s")
    return pl.kernel(
        _dispatch_body,
        mesh=mesh,
        out_type=jax.ShapeDtypeStruct((_PMAX, _D), jnp.float32),
        scratch_types=[
            pltpu.VMEM((_W_ROWS, _D), jnp.float32),
            pltpu.VMEM((_W_ROWS,), jnp.int32),
            pltpu.VMEM((_W_ROWS,), jnp.int32),
            pltpu.SemaphoreType.DMA,
            pltpu.SemaphoreType.DMA,
            pltpu.SemaphoreType.DMA,
        ],
    )(x, pos)


def _ffn_body(te_ref, va_ref, xs_ref, g_ref, u_ref, d_ref, ys_ref):
    i = pl.program_id(0)

    @pl.when(va_ref[0, i] > 0)
    def _():
        x = xs_ref[...].astype(jnp.bfloat16)
        g = g_ref[0].astype(jnp.bfloat16)
        u = u_ref[0].astype(jnp.bfloat16)
        d = d_ref[0].astype(jnp.bfloat16)
        a = jnp.dot(x, g.T, preferred_element_type=jnp.float32)
        b = jnp.dot(x, u.T, preferred_element_type=jnp.float32)
        h = (a * jax.nn.sigmoid(a)) * b
        ys_ref[...] = jnp.dot(h.astype(jnp.bfloat16), d.T,
                              preferred_element_type=jnp.float32)


def _tc_ffn(te, valid, xs, gate, up, down):
    grid_spec = pltpu.PrefetchScalarGridSpec(
        num_scalar_prefetch=2,
        grid=(_NT,),
        in_specs=[
            pl.BlockSpec((_BT, _D), lambda i, te, va: (i, 0)),
            pl.BlockSpec((1, _DFF, _D), lambda i, te, va: (te[0, i], 0, 0)),
            pl.BlockSpec((1, _DFF, _D), lambda i, te, va: (te[0, i], 0, 0)),
            pl.BlockSpec((1, _D, _DFF), lambda i, te, va: (te[0, i], 0, 0)),
        ],
        out_specs=pl.BlockSpec((_BT, _D), lambda i, te, va: (i, 0)),
    )
    return pl.pallas_call(
        _ffn_body,
        grid_spec=grid_spec,
        out_shape=jax.ShapeDtypeStruct((_PMAX, _D), jnp.float32),
    )(te, valid, xs, gate, up, down)


def _combine_body(ys_hbm, pos_hbm, w_hbm, out_hbm, i0_v, i1_v, w0_v, w1_v,
                  r0_v, r1_v, s0, s1, s2):
    wid = lax.axis_index("s") * _NC + lax.axis_index("c")
    b = wid * _W_ROWS
    l0 = pltpu.async_copy(pos_hbm.at[pl.ds(b, _W_ROWS)], i0_v, s0)
    l1 = pltpu.async_copy(pos_hbm.at[pl.ds(_T + b, _W_ROWS)], i1_v, s1)
    lw0 = pltpu.async_copy(w_hbm.at[pl.ds(b, _W_ROWS)], w0_v, s2)
    lw1 = pltpu.async_copy(w_hbm.at[pl.ds(_T + b, _W_ROWS)], w1_v, s2)
    l0.wait()
    l1.wait()
    cp0 = pltpu.async_copy(ys_hbm.at[i0_v], r0_v, s0)
    cp1 = pltpu.async_copy(ys_hbm.at[i1_v], r1_v, s1)
    lw0.wait()
    lw1.wait()
    cp0.wait()
    cp1.wait()

    def row_fn(r, carry):
        blk = (r // 16) * 16
        lane = jnp.full((16,), r - blk, jnp.int32)
        w0 = w0_v[pl.ds(blk, 16)].at[lane].get(mode="promise_in_bounds")
        w1 = w1_v[pl.ds(blk, 16)].at[lane].get(mode="promise_in_bounds")
        for cc in range(_D // 16):
            sl = pl.ds(cc * 16, 16)
            r0_v[r, sl] = r0_v[r, sl] * w0 + r1_v[r, sl] * w1
        return carry

    lax.fori_loop(0, _W_ROWS, row_fn, 0)
    pltpu.sync_copy(r0_v, out_hbm.at[pl.ds(b, _W_ROWS)])


def _sc_combine(ys, pos, w_kmaj):
    mesh = plsc.VectorSubcoreMesh(core_axis_name="c", subcore_axis_name="s")
    return pl.kernel(
        _combine_body,
        mesh=mesh,
        out_type=jax.ShapeDtypeStruct((_T, _D), jnp.float32),
        scratch_types=[
            pltpu.VMEM((_W_ROWS,), jnp.int32),
            pltpu.VMEM((_W_ROWS,), jnp.int32),
            pltpu.VMEM((_W_ROWS,), jnp.float32),
            pltpu.VMEM((_W_ROWS,), jnp.float32),
            pltpu.VMEM((_W_ROWS, _D), jnp.float32),
            pltpu.VMEM((_W_ROWS, _D), jnp.float32),
            pltpu.SemaphoreType.DMA,
            pltpu.SemaphoreType.DMA,
            pltpu.SemaphoreType.DMA,
        ],
    )(ys, pos, w_kmaj)


def kernel(hidden_states, topk_ids, topk_weights, gate_proj, up_proj,
           down_proj):
    B, S, D = hidden_states.shape
    x = hidden_states.reshape(B * S, D)
    ids_grid = topk_ids.astype(jnp.int32).T.reshape(_RR, 128)
    w_kmaj = topk_weights.astype(jnp.float32).T.reshape(-1)

    pos_grid, te, valid = _tc_route(ids_grid)
    pos = pos_grid.reshape(_TK)

    xs = _sc_dispatch(x, pos)
    ys = _tc_ffn(te, valid, xs, gate_proj, up_proj, down_proj)
    out = _sc_combine(ys, pos, w_kmaj)
    return out.reshape(B, S, D)


# speedup vs baseline: 1.4078x; 1.4078x over previous
import functools

import jax
import jax.numpy as jnp
from jax import lax
from jax.experimental import pallas as pl
from jax.experimental.pallas import tpu as pltpu
from jax.experimental.pallas import tpu_sc as plsc

_E = 8
_K = 2
_D = 768
_DFF = 2048
_T = 2048
_TK = _T * _K

_BT = 256
_NT = _TK // _BT + _E
_PMAX = _NT * _BT

_NC, _NS = 2, 16
_NW = _NC * _NS

_W_ROWS = _T // _NW
_SLOTS_W = _K * _W_ROWS

_RR = _TK // 128


def _route_body(ids_ref, pos_ref, te_ref, va_ref):
    ids = ids_ref[...]
    jj = lax.broadcasted_iota(jnp.int32, (128, 128), 0)
    ll = lax.broadcasted_iota(jnp.int32, (128, 128), 1)
    tri_lane = (jj <= ll).astype(jnp.float32)
    ii = lax.broadcasted_iota(jnp.int32, (_RR, _RR), 0)
    kk = lax.broadcasted_iota(jnp.int32, (_RR, _RR), 1)
    tri_row = (kk < ii).astype(jnp.float32)

    ohs, csums, counts = [], [], []
    for e in range(_E):
        oh = (ids == e).astype(jnp.float32)
        lane_cs = jnp.dot(oh, tri_lane, preferred_element_type=jnp.float32)
        row_tot = lane_cs[:, 127:128]
        row_pre = jnp.dot(tri_row, row_tot, preferred_element_type=jnp.float32)
        ohs.append(oh)
        csums.append(lane_cs + row_pre)
        counts.append(jnp.sum(row_tot))

    poffs, cps = [], []
    cum = jnp.float32(0.0)
    for e in range(_E):
        poffs.append(cum)
        cum = cum + jnp.ceil(counts[e] * (1.0 / _BT)) * _BT
        cps.append(cum)

    posf = jnp.zeros((_RR, 128), jnp.float32)
    for e in range(_E):
        posf = posf + ohs[e] * (poffs[e] + csums[e] - 1.0)
    pos_ref[...] = posf.astype(jnp.int32)

    starts = (lax.broadcasted_iota(jnp.int32, (1, 128), 1)
              .astype(jnp.float32) * float(_BT))
    te = jnp.zeros((1, 128), jnp.int32)
    for e in range(_E):
        te = te + (starts >= cps[e]).astype(jnp.int32)
    te_ref[...] = jnp.minimum(te, _E - 1)
    va_ref[...] = (starts < cum).astype(jnp.int32)


def _tc_route(ids_grid):
    return pl.pallas_call(
        _route_body,
        out_shape=(
            jax.ShapeDtypeStruct((_RR, 128), jnp.int32),
            jax.ShapeDtypeStruct((1, 128), jnp.int32),
            jax.ShapeDtypeStruct((1, 128), jnp.int32),
        ),
    )(ids_grid)


def _dispatch_body(x_hbm, pos_hbm, xs_hbm, xv, i0_v, i1_v, s0, s1, s2):
    wid = lax.axis_index("s") * _NC + lax.axis_index("c")
    b = wid * _W_ROWS
    l0 = pltpu.async_copy(pos_hbm.at[pl.ds(b, _W_ROWS)], i0_v, s0)
    l1 = pltpu.async_copy(pos_hbm.at[pl.ds(_T + b, _W_ROWS)], i1_v, s1)
    lx = pltpu.async_copy(x_hbm.at[pl.ds(b, _W_ROWS)], xv, s2)
    l0.wait()
    l1.wait()
    lx.wait()
    c0 = pltpu.async_copy(xv, xs_hbm.at[i0_v], s0)
    c1 = pltpu.async_copy(xv, xs_hbm.at[i1_v], s1)
    c0.wait()
    c1.wait()


def _sc_dispatch(x, pos):
    mesh = plsc.VectorSubcoreMesh(core_axis_name="c", subcore_axis_name="s")
    return pl.kernel(
        _dispatch_body,
        mesh=mesh,
        out_type=jax.ShapeDtypeStruct((_PMAX, _D), jnp.float32),
        scratch_types=[
            pltpu.VMEM((_W_ROWS, _D), jnp.float32),
            pltpu.VMEM((_W_ROWS,), jnp.int32),
            pltpu.VMEM((_W_ROWS,), jnp.int32),
            pltpu.SemaphoreType.DMA,
            pltpu.SemaphoreType.DMA,
            pltpu.SemaphoreType.DMA,
        ],
    )(x, pos)


def _ffn_body(te_ref, va_ref, xs_ref, g_ref, u_ref, d_ref, ys_ref):
    i = pl.program_id(0)

    @pl.when(va_ref[0, i] > 0)
    def _():
        x = xs_ref[...].astype(jnp.bfloat16)
        g = g_ref[0].astype(jnp.bfloat16)
        u = u_ref[0].astype(jnp.bfloat16)
        d = d_ref[0].astype(jnp.bfloat16)
        a = jnp.dot(x, g.T, preferred_element_type=jnp.float32)
        b = jnp.dot(x, u.T, preferred_element_type=jnp.float32)
        h = (a * jax.nn.sigmoid(a)) * b
        ys_ref[...] = jnp.dot(h.astype(jnp.bfloat16), d.T,
                              preferred_element_type=jnp.float32)


def _tc_ffn(te, valid, xs, gate, up, down):
    grid_spec = pltpu.PrefetchScalarGridSpec(
        num_scalar_prefetch=2,
        grid=(_NT,),
        in_specs=[
            pl.BlockSpec((_BT, _D), lambda i, te, va: (i, 0)),
            pl.BlockSpec((1, _DFF, _D), lambda i, te, va: (te[0, i], 0, 0)),
            pl.BlockSpec((1, _DFF, _D), lambda i, te, va: (te[0, i], 0, 0)),
            pl.BlockSpec((1, _D, _DFF), lambda i, te, va: (te[0, i], 0, 0)),
        ],
        out_specs=pl.BlockSpec((_BT, _D), lambda i, te, va: (i, 0)),
    )
    return pl.pallas_call(
        _ffn_body,
        grid_spec=grid_spec,
        out_shape=jax.ShapeDtypeStruct((_PMAX, _D), jnp.float32),
    )(te, valid, xs, gate, up, down)


def _combine_body(ys_hbm, pos_hbm, w_hbm, out_hbm, i0_v, i1_v, w0_v, w1_v,
                  r0_v, r1_v, s0, s1, s2):
    wid = lax.axis_index("s") * _NC + lax.axis_index("c")
    b = wid * _W_ROWS
    l0 = pltpu.async_copy(pos_hbm.at[pl.ds(b, _W_ROWS)], i0_v, s0)
    l1 = pltpu.async_copy(pos_hbm.at[pl.ds(_T + b, _W_ROWS)], i1_v, s1)
    lw0 = pltpu.async_copy(w_hbm.at[pl.ds(b, _W_ROWS)], w0_v, s2)
    lw1 = pltpu.async_copy(w_hbm.at[pl.ds(_T + b, _W_ROWS)], w1_v, s2)
    l0.wait()
    l1.wait()
    cp0 = pltpu.async_copy(ys_hbm.at[i0_v], r0_v, s0)
    cp1 = pltpu.async_copy(ys_hbm.at[i1_v], r1_v, s1)
    lw0.wait()
    lw1.wait()
    cp0.wait()
    cp1.wait()

    def row_fn(r, carry):
        blk = (r // 16) * 16
        lane = jnp.full((16,), r - blk, jnp.int32)
        w0 = w0_v[pl.ds(blk, 16)].at[lane].get(mode="promise_in_bounds")
        w1 = w1_v[pl.ds(blk, 16)].at[lane].get(mode="promise_in_bounds")
        for cc in range(_D // 16):
            sl = pl.ds(cc * 16, 16)
            r0_v[r, sl] = r0_v[r, sl] * w0 + r1_v[r, sl] * w1
        return carry

    lax.fori_loop(0, _W_ROWS, row_fn, 0)
    pltpu.sync_copy(r0_v, out_hbm.at[pl.ds(b, _W_ROWS)])


def _sc_combine(ys, pos, w_kmaj):
    mesh = plsc.VectorSubcoreMesh(core_axis_name="c", subcore_axis_name="s")
    return pl.kernel(
        _combine_body,
        mesh=mesh,
        out_type=jax.ShapeDtypeStruct((_T, _D), jnp.float32),
        scratch_types=[
            pltpu.VMEM((_W_ROWS,), jnp.int32),
            pltpu.VMEM((_W_ROWS,), jnp.int32),
            pltpu.VMEM((_W_ROWS,), jnp.float32),
            pltpu.VMEM((_W_ROWS,), jnp.float32),
            pltpu.VMEM((_W_ROWS, _D), jnp.float32),
            pltpu.VMEM((_W_ROWS, _D), jnp.float32),
            pltpu.SemaphoreType.DMA,
            pltpu.SemaphoreType.DMA,
            pltpu.SemaphoreType.DMA,
        ],
    )(ys, pos, w_kmaj)


def kernel(hidden_states, topk_ids, topk_weights, gate_proj, up_proj,
           down_proj):
    B, S, D = hidden_states.shape
    x = hidden_states.reshape(B * S, D)
    ids_grid = topk_ids.astype(jnp.int32).T.reshape(_RR, 128)
    w_kmaj = topk_weights.astype(jnp.float32).T.reshape(-1)

    pos_grid, te, valid = _tc_route(ids_grid)
    pos = pos_grid.reshape(_TK)

    xs = _sc_dispatch(x, pos)
    ys = _tc_ffn(te, valid, xs, gate_proj, up_proj, down_proj)
    out = _sc_combine(ys, pos, w_kmaj)
    return out.reshape(B, S, D)
